# baseline (speedup 1.0000x reference)
import math
import functools

import jax
import jax.numpy as jnp
from jax import lax
from jax.experimental import pallas as pl
from jax.experimental.pallas import tpu as pltpu
from jax.experimental.pallas import tpu_sc as plsc

_N = 10000
_F = 128
_R = 20
_CUT = 5.0
_L = 3
_E = 262144
_NP = 10240
_RB = 256
_CB = 1024


def _mask_kernel(pr, pc, m_ref, cnt_ref):
    i = pl.program_id(0)
    j = pl.program_id(1)

    @pl.when((i == 0) & (j == 0))
    def _():
        cnt_ref[0, 0] = 0

    xr, yr, zr = pr[0, :], pr[1, :], pr[2, :]
    xc, yc, zc = pc[0, :], pc[1, :], pc[2, :]

    def _b(t):
        return t.astype(jnp.bfloat16).astype(jnp.float32)

    cross = (_b(xr)[:, None] * _b(xc)[None, :]
             + _b(yr)[:, None] * _b(yc)[None, :]
             + _b(zr)[:, None] * _b(zc)[None, :])
    sqr = xr * xr + yr * yr + zr * zr
    sqc = xc * xc + yc * yc + zc * zc
    d2 = sqr[:, None] + sqc[None, :] - 2.0 * cross
    rid = i * _RB + jax.lax.broadcasted_iota(jnp.int32, (_RB, _CB), 0)
    cid = j * _CB + jax.lax.broadcasted_iota(jnp.int32, (_RB, _CB), 1)
    m = (d2 < _CUT * _CUT) & (rid != cid) & (rid < _N) & (cid < _N)
    m_ref[...] = m.astype(jnp.int8)
    cnt_ref[0, 0] += jnp.sum(m.astype(jnp.int32))


def _radius_graph(pos):
    posT = jnp.zeros((8, _NP), jnp.float32).at[:3, :_N].set(pos.T)
    mask, cnt = pl.pallas_call(
        _mask_kernel,
        grid=(_NP // _RB, _NP // _CB),
        in_specs=[
            pl.BlockSpec((8, _RB), lambda i, j: (0, i)),
            pl.BlockSpec((8, _CB), lambda i, j: (0, j)),
        ],
        out_specs=[
            pl.BlockSpec((_RB, _CB), lambda i, j: (i, j)),
            pl.BlockSpec(memory_space=pltpu.SMEM),
        ],
        out_shape=[
            jax.ShapeDtypeStruct((_NP, _NP), jnp.int8),
            jax.ShapeDtypeStruct((1, 1), jnp.int32),
        ],
    )(posT, posT)
    dst, src = jnp.nonzero(mask, size=_E, fill_value=0)
    count = cnt[0, 0]
    valid = jnp.arange(_E) < count
    return dst.astype(jnp.int32), src.astype(jnp.int32), valid


_NC = 2
_NS = 16
_NW = _NC * _NS


@functools.lru_cache(maxsize=None)
def _sc_gather_fn(E, D, CH):
    EPW = E // _NW
    mesh = plsc.VectorSubcoreMesh(core_axis_name="c", subcore_axis_name="s")

    @functools.partial(
        pl.kernel,
        mesh=mesh,
        out_type=jax.ShapeDtypeStruct((E, D), jnp.float32),
        scratch_types=[
            pltpu.VMEM((EPW,), jnp.int32),
            pltpu.VMEM((CH, D), jnp.float32),
            pltpu.SemaphoreType.DMA,
        ],
    )
    def k(table_hbm, idx_hbm, out_hbm, idx_v, rows_v, sem):
        wid = lax.axis_index("s") * _NC + lax.axis_index("c")
        base = wid * EPW
        pltpu.sync_copy(idx_hbm.at[pl.ds(base, EPW)], idx_v)

        def body(c, carry):
            a = c * CH
            pltpu.async_copy(
                table_hbm.at[idx_v.at[pl.ds(a, CH)]], rows_v, sem
            ).wait()
            pltpu.sync_copy(rows_v, out_hbm.at[pl.ds(base + a, CH)])
            return carry

        lax.fori_loop(0, EPW // CH, body, 0)

    return k


def _sc_gather(table, idx, CH=128):
    return _sc_gather_fn(idx.shape[0], table.shape[1], CH)(table, idx)


def _silu(x):
    return x * jax.nn.sigmoid(x)


def _bessel(d):
    freqs = jnp.arange(1, _R + 1, dtype=jnp.float32) * math.pi / _CUT
    ax = d[:, None] * freqs[None, :]
    norm = jnp.where(d == 0, 1.0, d)
    return jnp.sin(ax) / norm[:, None]


def _cutoff_fn(d):
    return 0.5 * (jnp.cos(d * math.pi / _CUT) + 1.0) * (d < _CUT).astype(jnp.float32)


def kernel(z, pos, params):
    dst, src, valid = _radius_graph(pos)

    s = params["embedding"][z]
    v = jnp.zeros((_N, _F, 3), dtype=jnp.float32)
    posp = jnp.zeros((_N, 128), jnp.float32).at[:, :3].set(pos)
    p_dst = _sc_gather(posp, dst, CH=512)[:, :3]
    p_src = _sc_gather(posp, src, CH=512)[:, :3]
    rij = p_dst - p_src
    d = jnp.linalg.norm(rij, axis=1)
    rbf = _bessel(d)
    cut = _cutoff_fn(d)
    rn = rij / jnp.maximum(d[:, None], 1e-12)

    for L in range(_L):
        mp = params["msg"][L]
        up = params["upd"][L]

        phi_n = _silu(s @ mp["lin1"]["W"] + mp["lin1"]["b"])
        phi_n = phi_n @ mp["lin2"]["W"] + mp["lin2"]["b"]
        W = (rbf @ mp["lin_rbf"]["W"] + mp["lin_rbf"]["b"]) * cut[:, None]
        phi_g = _sc_gather(phi_n, src)
        pw = phi_g * W
        pw = jnp.where(valid[:, None], pw, 0.0)
        left = pw[:, :_F]
        dsm = pw[:, _F:2 * _F]
        right = pw[:, 2 * _F:]
        v_g = _sc_gather(v.reshape(_N, 3 * _F), src).reshape(_E, _F, 3)
        dvm = v_g * left[:, :, None] + right[:, :, None] * rn[:, None, :]
        ds = jax.ops.segment_sum(dsm, dst, num_segments=_N)
        dv = jax.ops.segment_sum(dvm, dst, num_segments=_N)
        s = ds + s
        v = dv + v

        v_ut = jnp.swapaxes(v, 1, 2)
        U_v = jnp.swapaxes(v_ut @ up["denseU"]["W"], 1, 2)
        V_v = jnp.swapaxes(v_ut @ up["denseV"]["W"], 1, 2)
        dot = jnp.sum(U_v * V_v, axis=-1)
        V_norm = jnp.sqrt(jnp.sum(V_v * V_v, axis=-1) + 1e-12)
        a = jnp.concatenate([s, V_norm], axis=-1)
        a = _silu(a @ up["lin_up"]["W"] + up["lin_up"]["b"])
        a = a @ up["lin2"]["W"] + up["lin2"]["b"]
        a_vv = a[:, :_F]
        a_sv = a[:, _F:2 * _F]
        a_ss = a[:, 2 * _F:]
        s = s + a_ss + a_sv * dot
        v = v + U_v * a_vv[:, :, None]

    W = params["lin"]["W"]
    b = params["lin"]["b"]
    s = _silu(s @ W + b)
    s = s @ W + b
    return s


# speedup vs baseline: 3.6036x; 3.6036x over previous
import math
import functools

import jax
import jax.numpy as jnp
from jax import lax
from jax.experimental import pallas as pl
from jax.experimental.pallas import tpu as pltpu
from jax.experimental.pallas import tpu_sc as plsc

_N = 10000
_F = 128
_R = 20
_CUT = 5.0
_L = 3
_E = 262144
_NP = 10240
_RB = 256
_CB = 1024


def _mask_kernel(pr, pc, m_ref, cnt_ref):
    i = pl.program_id(0)
    j = pl.program_id(1)

    @pl.when((i == 0) & (j == 0))
    def _():
        cnt_ref[0, 0] = 0

    xr, yr, zr = pr[0, :], pr[1, :], pr[2, :]
    xc, yc, zc = pc[0, :], pc[1, :], pc[2, :]

    def _b(t):
        return t.astype(jnp.bfloat16).astype(jnp.float32)

    cross = (_b(xr)[:, None] * _b(xc)[None, :]
             + _b(yr)[:, None] * _b(yc)[None, :]
             + _b(zr)[:, None] * _b(zc)[None, :])
    sqr = xr * xr + yr * yr + zr * zr
    sqc = xc * xc + yc * yc + zc * zc
    d2 = sqr[:, None] + sqc[None, :] - 2.0 * cross
    rid = i * _RB + jax.lax.broadcasted_iota(jnp.int32, (_RB, _CB), 0)
    cid = j * _CB + jax.lax.broadcasted_iota(jnp.int32, (_RB, _CB), 1)
    m = (d2 < _CUT * _CUT) & (rid != cid) & (rid < _N) & (cid < _N)
    m_ref[...] = m.astype(jnp.int8)
    cnt_ref[0, 0] += jnp.sum(m.astype(jnp.int32))


def _radius_graph(pos):
    posT = jnp.zeros((8, _NP), jnp.float32).at[:3, :_N].set(pos.T)
    mask, cnt = pl.pallas_call(
        _mask_kernel,
        grid=(_NP // _RB, _NP // _CB),
        in_specs=[
            pl.BlockSpec((8, _RB), lambda i, j: (0, i)),
            pl.BlockSpec((8, _CB), lambda i, j: (0, j)),
        ],
        out_specs=[
            pl.BlockSpec((_RB, _CB), lambda i, j: (i, j)),
            pl.BlockSpec(memory_space=pltpu.SMEM),
        ],
        out_shape=[
            jax.ShapeDtypeStruct((_NP, _NP), jnp.int8),
            jax.ShapeDtypeStruct((1, 1), jnp.int32),
        ],
    )(posT, posT)
    dst, src = jnp.nonzero(mask, size=_E, fill_value=0)
    count = cnt[0, 0]
    valid = jnp.arange(_E) < count
    return dst.astype(jnp.int32), src.astype(jnp.int32), valid


_NC = 2
_NS = 16
_NW = _NC * _NS


@functools.lru_cache(maxsize=None)
def _sc_gather_fn(E, D, CH):
    EPW = E // _NW
    mesh = plsc.VectorSubcoreMesh(core_axis_name="c", subcore_axis_name="s")

    @functools.partial(
        pl.kernel,
        mesh=mesh,
        out_type=jax.ShapeDtypeStruct((E, D), jnp.float32),
        scratch_types=[
            pltpu.VMEM((EPW,), jnp.int32),
            pltpu.VMEM((CH, D), jnp.float32),
            pltpu.SemaphoreType.DMA,
        ],
    )
    def k(table_hbm, idx_hbm, out_hbm, idx_v, rows_v, sem):
        wid = lax.axis_index("s") * _NC + lax.axis_index("c")
        base = wid * EPW
        pltpu.sync_copy(idx_hbm.at[pl.ds(base, EPW)], idx_v)

        def body(c, carry):
            a = c * CH
            pltpu.async_copy(
                table_hbm.at[idx_v.at[pl.ds(a, CH)]], rows_v, sem
            ).wait()
            pltpu.sync_copy(rows_v, out_hbm.at[pl.ds(base + a, CH)])
            return carry

        lax.fori_loop(0, EPW // CH, body, 0)

    return k


def _sc_gather(table, idx, CH=128):
    return _sc_gather_fn(idx.shape[0], table.shape[1], CH)(table, idx)


def _silu(x):
    return x * jax.nn.sigmoid(x)


def _bessel(d):
    freqs = jnp.arange(1, _R + 1, dtype=jnp.float32) * math.pi / _CUT
    ax = d[:, None] * freqs[None, :]
    norm = jnp.where(d == 0, 1.0, d)
    return jnp.sin(ax) / norm[:, None]


def _cutoff_fn(d):
    return 0.5 * (jnp.cos(d * math.pi / _CUT) + 1.0) * (d < _CUT).astype(jnp.float32)


def kernel(z, pos, params):
    dst, src, valid = _radius_graph(pos)

    s = params["embedding"][z]
    v = jnp.zeros((3, _N, _F), dtype=jnp.float32)
    rij = pos[dst] - pos[src]
    d = jnp.linalg.norm(rij, axis=1)
    rbf = _bessel(d)
    cut = _cutoff_fn(d)
    rn = rij / jnp.maximum(d[:, None], 1e-12)

    for L in range(_L):
        mp = params["msg"][L]
        up = params["upd"][L]

        phi_n = _silu(s @ mp["lin1"]["W"] + mp["lin1"]["b"])
        phi_n = phi_n @ mp["lin2"]["W"] + mp["lin2"]["b"]
        W = (rbf @ mp["lin_rbf"]["W"] + mp["lin_rbf"]["b"]) * cut[:, None]
        pw = phi_n[src] * W
        pw = jnp.where(valid[:, None], pw, 0.0)
        left = pw[:, :_F]
        dsm = pw[:, _F:2 * _F]
        right = pw[:, 2 * _F:]
        ds = jax.ops.segment_sum(dsm, dst, num_segments=_N)
        dv = [
            jax.ops.segment_sum(
                v[c][src] * left + right * rn[:, c:c + 1],
                dst, num_segments=_N)
            for c in range(3)
        ]
        s = ds + s
        v = v + jnp.stack(dv, axis=0)

        U_v = jnp.einsum("cnf,fg->cng", v, up["denseU"]["W"])
        V_v = jnp.einsum("cnf,fg->cng", v, up["denseV"]["W"])
        dot = jnp.sum(U_v * V_v, axis=0)
        V_norm = jnp.sqrt(jnp.sum(V_v * V_v, axis=0) + 1e-12)
        a = jnp.concatenate([s, V_norm], axis=-1)
        a = _silu(a @ up["lin_up"]["W"] + up["lin_up"]["b"])
        a = a @ up["lin2"]["W"] + up["lin2"]["b"]
        a_vv = a[:, :_F]
        a_sv = a[:, _F:2 * _F]
        a_ss = a[:, 2 * _F:]
        s = s + a_ss + a_sv * dot
        v = v + U_v * a_vv[None, :, :]

    W = params["lin"]["W"]
    b = params["lin"]["b"]
    s = _silu(s @ W + b)
    s = s @ W + b
    return s


# speedup vs baseline: 4.1662x; 1.1561x over previous
import math
import functools

import jax
import jax.numpy as jnp
from jax import lax
from jax.experimental import pallas as pl
from jax.experimental.pallas import tpu as pltpu
from jax.experimental.pallas import tpu_sc as plsc

_N = 10000
_F = 128
_R = 20
_CUT = 5.0
_L = 3
_E = 262144
_NP = 10240
_RB = 256
_CB = 1024


def _mask_kernel(pr, pc, m_ref, cnt_ref):
    i = pl.program_id(0)
    j = pl.program_id(1)

    @pl.when((i == 0) & (j == 0))
    def _():
        cnt_ref[0, 0] = 0

    xr, yr, zr = pr[0, :], pr[1, :], pr[2, :]
    xc, yc, zc = pc[0, :], pc[1, :], pc[2, :]

    def _b(t):
        return t.astype(jnp.bfloat16).astype(jnp.float32)

    cross = (_b(xr)[:, None] * _b(xc)[None, :]
             + _b(yr)[:, None] * _b(yc)[None, :]
             + _b(zr)[:, None] * _b(zc)[None, :])
    sqr = xr * xr + yr * yr + zr * zr
    sqc = xc * xc + yc * yc + zc * zc
    d2 = sqr[:, None] + sqc[None, :] - 2.0 * cross
    rid = i * _RB + jax.lax.broadcasted_iota(jnp.int32, (_RB, _CB), 0)
    cid = j * _CB + jax.lax.broadcasted_iota(jnp.int32, (_RB, _CB), 1)
    m = (d2 < _CUT * _CUT) & (rid != cid) & (rid < _N) & (cid < _N)
    m_ref[...] = m.astype(jnp.int8)
    cnt_ref[0, 0] += jnp.sum(m.astype(jnp.int32))


def _radius_graph(pos):
    posT = jnp.zeros((8, _NP), jnp.float32).at[:3, :_N].set(pos.T)
    mask, cnt = pl.pallas_call(
        _mask_kernel,
        grid=(_NP // _RB, _NP // _CB),
        in_specs=[
            pl.BlockSpec((8, _RB), lambda i, j: (0, i)),
            pl.BlockSpec((8, _CB), lambda i, j: (0, j)),
        ],
        out_specs=[
            pl.BlockSpec((_RB, _CB), lambda i, j: (i, j)),
            pl.BlockSpec(memory_space=pltpu.SMEM),
        ],
        out_shape=[
            jax.ShapeDtypeStruct((_NP, _NP), jnp.int8),
            jax.ShapeDtypeStruct((1, 1), jnp.int32),
        ],
    )(posT, posT)
    dst, src = jnp.nonzero(mask, size=_E, fill_value=0)
    count = cnt[0, 0]
    valid = jnp.arange(_E) < count
    return dst.astype(jnp.int32), src.astype(jnp.int32), valid


_NC = 2
_NS = 16
_NW = _NC * _NS


@functools.lru_cache(maxsize=None)
def _sc_gather_fn(E, D, CH):
    EPW = E // _NW
    mesh = plsc.VectorSubcoreMesh(core_axis_name="c", subcore_axis_name="s")

    @functools.partial(
        pl.kernel,
        mesh=mesh,
        out_type=jax.ShapeDtypeStruct((E, D), jnp.float32),
        scratch_types=[
            pltpu.VMEM((EPW,), jnp.int32),
            pltpu.VMEM((CH, D), jnp.float32),
            pltpu.SemaphoreType.DMA,
        ],
    )
    def k(table_hbm, idx_hbm, out_hbm, idx_v, rows_v, sem):
        wid = lax.axis_index("s") * _NC + lax.axis_index("c")
        base = wid * EPW
        pltpu.sync_copy(idx_hbm.at[pl.ds(base, EPW)], idx_v)

        def body(c, carry):
            a = c * CH
            pltpu.async_copy(
                table_hbm.at[idx_v.at[pl.ds(a, CH)]], rows_v, sem
            ).wait()
            pltpu.sync_copy(rows_v, out_hbm.at[pl.ds(base + a, CH)])
            return carry

        lax.fori_loop(0, EPW // CH, body, 0)

    return k


def _sc_gather(table, idx, CH=128):
    return _sc_gather_fn(idx.shape[0], table.shape[1], CH)(table, idx)


_CH = 128
_EPW = _E // _NW
_NCHK = _EPW // _CH
_ROWS = _NP // _NS


@functools.lru_cache(maxsize=None)
def _sc_segsum4_fn():
    mesh = plsc.VectorSubcoreMesh(core_axis_name="c", subcore_axis_name="s")

    @functools.partial(
        pl.kernel,
        mesh=mesh,
        out_type=jax.ShapeDtypeStruct((4, 2, _NP, _F), jnp.float32),
        scratch_types=[
            pltpu.VMEM((_NCHK, _CH), jnp.int32),
            pltpu.VMEM((_CH, _F), jnp.float32),
            pltpu.VMEM_SHARED((_NP, _F), jnp.float32),
        ],
    )
    def k(d0, d1, d2, d3, idx2, zeros_hbm, out_hbm, idx_v, data_v, acc):
        cid = lax.axis_index("c")
        sid = lax.axis_index("s")
        wid = cid * _NS + sid
        base = pl.multiple_of(wid * _EPW, _CH * 8)
        rowo = pl.multiple_of(sid * _ROWS, 8)
        pltpu.sync_copy(idx2.at[pl.ds(pl.multiple_of(base // _CH, 8), _NCHK)],
                        idx_v)

        for t, d in enumerate((d0, d1, d2, d3)):
            pltpu.sync_copy(zeros_hbm.at[pl.ds(rowo, _ROWS)],
                            acc.at[pl.ds(rowo, _ROWS)])
            plsc.subcore_barrier()

            def body(cc, carry, d=d):
                a = pl.multiple_of(base + cc * _CH, _CH)
                pltpu.sync_copy(d.at[pl.ds(a, _CH)], data_v)
                pltpu.sync_copy(data_v, acc.at[idx_v.at[cc]], add=True)
                return carry

            lax.fori_loop(0, _NCHK, body, 0)
            plsc.subcore_barrier()
            pltpu.sync_copy(acc.at[pl.ds(rowo, _ROWS)],
                            out_hbm.at[t, cid, pl.ds(rowo, _ROWS)])
            plsc.subcore_barrier()

    return k


def _sc_segsum4(d0, d1, d2, d3, dst):
    idx2 = dst.reshape(_E // _CH, _CH)
    zeros = jnp.zeros((_NP, _F), jnp.float32)
    out = _sc_segsum4_fn()(d0, d1, d2, d3, idx2, zeros)
    r = out[:, 0] + out[:, 1]
    return r[0, :_N], r[1, :_N], r[2, :_N], r[3, :_N]


def _silu(x):
    return x * jax.nn.sigmoid(x)


def _bessel(d):
    freqs = jnp.arange(1, _R + 1, dtype=jnp.float32) * math.pi / _CUT
    ax = d[:, None] * freqs[None, :]
    norm = jnp.where(d == 0, 1.0, d)
    return jnp.sin(ax) / norm[:, None]


def _cutoff_fn(d):
    return 0.5 * (jnp.cos(d * math.pi / _CUT) + 1.0) * (d < _CUT).astype(jnp.float32)


def kernel(z, pos, params):
    dst, src, valid = _radius_graph(pos)

    s = params["embedding"][z]
    v = jnp.zeros((3, _N, _F), dtype=jnp.float32)
    rij = pos[dst] - pos[src]
    d = jnp.linalg.norm(rij, axis=1)
    rbf = _bessel(d)
    cut = _cutoff_fn(d)
    rn = rij / jnp.maximum(d[:, None], 1e-12)

    for L in range(_L):
        mp = params["msg"][L]
        up = params["upd"][L]

        phi_n = _silu(s @ mp["lin1"]["W"] + mp["lin1"]["b"])
        phi_n = phi_n @ mp["lin2"]["W"] + mp["lin2"]["b"]
        W = (rbf @ mp["lin_rbf"]["W"] + mp["lin_rbf"]["b"]) * cut[:, None]
        pw = phi_n[src] * W
        pw = jnp.where(valid[:, None], pw, 0.0)
        left = pw[:, :_F]
        dsm = pw[:, _F:2 * _F]
        right = pw[:, 2 * _F:]
        dvm = [v[c][src] * left + right * rn[:, c:c + 1] for c in range(3)]
        ds, dv0, dv1, dv2 = _sc_segsum4(dsm, dvm[0], dvm[1], dvm[2], dst)
        s = ds + s
        v = v + jnp.stack([dv0, dv1, dv2], axis=0)

        U_v = jnp.einsum("cnf,fg->cng", v, up["denseU"]["W"])
        V_v = jnp.einsum("cnf,fg->cng", v, up["denseV"]["W"])
        dot = jnp.sum(U_v * V_v, axis=0)
        V_norm = jnp.sqrt(jnp.sum(V_v * V_v, axis=0) + 1e-12)
        a = jnp.concatenate([s, V_norm], axis=-1)
        a = _silu(a @ up["lin_up"]["W"] + up["lin_up"]["b"])
        a = a @ up["lin2"]["W"] + up["lin2"]["b"]
        a_vv = a[:, :_F]
        a_sv = a[:, _F:2 * _F]
        a_ss = a[:, 2 * _F:]
        s = s + a_ss + a_sv * dot
        v = v + U_v * a_vv[None, :, :]

    W = params["lin"]["W"]
    b = params["lin"]["b"]
    s = _silu(s @ W + b)
    s = s @ W + b
    return s


# speedup vs baseline: 4.1673x; 1.0003x over previous
import math
import functools

import jax
import jax.numpy as jnp
from jax import lax
from jax.experimental import pallas as pl
from jax.experimental.pallas import tpu as pltpu
from jax.experimental.pallas import tpu_sc as plsc

_N = 10000
_F = 128
_R = 20
_CUT = 5.0
_L = 3
_E = 262144
_NP = 10240
_RB = 256
_CB = 1024


def _mask_kernel(pr, pc, m_ref, cnt_ref):
    i = pl.program_id(0)
    j = pl.program_id(1)

    @pl.when((i == 0) & (j == 0))
    def _():
        cnt_ref[0, 0] = 0

    xr, yr, zr = pr[0, :], pr[1, :], pr[2, :]
    xc, yc, zc = pc[0, :], pc[1, :], pc[2, :]

    def _b(t):
        return t.astype(jnp.bfloat16).astype(jnp.float32)

    cross = (_b(xr)[:, None] * _b(xc)[None, :]
             + _b(yr)[:, None] * _b(yc)[None, :]
             + _b(zr)[:, None] * _b(zc)[None, :])
    sqr = xr * xr + yr * yr + zr * zr
    sqc = xc * xc + yc * yc + zc * zc
    d2 = sqr[:, None] + sqc[None, :] - 2.0 * cross
    rid = i * _RB + jax.lax.broadcasted_iota(jnp.int32, (_RB, _CB), 0)
    cid = j * _CB + jax.lax.broadcasted_iota(jnp.int32, (_RB, _CB), 1)
    m = (d2 < _CUT * _CUT) & (rid != cid) & (rid < _N) & (cid < _N)
    m_ref[...] = m.astype(jnp.int8)
    cnt_ref[0, 0] += jnp.sum(m.astype(jnp.int32))


def _radius_graph(pos):
    posT = jnp.zeros((8, _NP), jnp.float32).at[:3, :_N].set(pos.T)
    mask, cnt = pl.pallas_call(
        _mask_kernel,
        grid=(_NP // _RB, _NP // _CB),
        in_specs=[
            pl.BlockSpec((8, _RB), lambda i, j: (0, i)),
            pl.BlockSpec((8, _CB), lambda i, j: (0, j)),
        ],
        out_specs=[
            pl.BlockSpec((_RB, _CB), lambda i, j: (i, j)),
            pl.BlockSpec(memory_space=pltpu.SMEM),
        ],
        out_shape=[
            jax.ShapeDtypeStruct((_NP, _NP), jnp.int8),
            jax.ShapeDtypeStruct((1, 1), jnp.int32),
        ],
    )(posT, posT)
    dst, src = jnp.nonzero(mask, size=_E, fill_value=0)
    count = cnt[0, 0]
    valid = jnp.arange(_E) < count
    return dst.astype(jnp.int32), src.astype(jnp.int32), valid, count


_NC = 2
_NS = 16
_NW = _NC * _NS


@functools.lru_cache(maxsize=None)
def _sc_gather_fn(E, D, CH):
    EPW = E // _NW
    mesh = plsc.VectorSubcoreMesh(core_axis_name="c", subcore_axis_name="s")

    @functools.partial(
        pl.kernel,
        mesh=mesh,
        out_type=jax.ShapeDtypeStruct((E, D), jnp.float32),
        scratch_types=[
            pltpu.VMEM((EPW,), jnp.int32),
            pltpu.VMEM((CH, D), jnp.float32),
            pltpu.SemaphoreType.DMA,
        ],
    )
    def k(table_hbm, idx_hbm, out_hbm, idx_v, rows_v, sem):
        wid = lax.axis_index("s") * _NC + lax.axis_index("c")
        base = wid * EPW
        pltpu.sync_copy(idx_hbm.at[pl.ds(base, EPW)], idx_v)

        def body(c, carry):
            a = c * CH
            pltpu.async_copy(
                table_hbm.at[idx_v.at[pl.ds(a, CH)]], rows_v, sem
            ).wait()
            pltpu.sync_copy(rows_v, out_hbm.at[pl.ds(base + a, CH)])
            return carry

        lax.fori_loop(0, EPW // CH, body, 0)

    return k


def _sc_gather(table, idx, CH=128):
    return _sc_gather_fn(idx.shape[0], table.shape[1], CH)(table, idx)


_CH = 128
_EPW = _E // _NW
_NCHK = _EPW // _CH
_ROWS = _NP // _NS


@functools.lru_cache(maxsize=None)
def _sc_segsum4_fn():
    mesh = plsc.VectorSubcoreMesh(core_axis_name="c", subcore_axis_name="s")

    @functools.partial(
        pl.kernel,
        mesh=mesh,
        out_type=jax.ShapeDtypeStruct((4, 2, _NP, _F), jnp.float32),
        scratch_types=[
            pltpu.VMEM((_NCHK, _CH), jnp.int32),
            pltpu.VMEM((_CH, _F), jnp.float32),
            pltpu.VMEM_SHARED((_NP, _F), jnp.float32),
        ],
    )
    def k(d0, d1, d2, d3, idx2, zeros_hbm, out_hbm, idx_v, data_v, acc):
        cid = lax.axis_index("c")
        sid = lax.axis_index("s")
        wid = cid * _NS + sid
        base = pl.multiple_of(wid * _EPW, _CH * 8)
        rowo = pl.multiple_of(sid * _ROWS, 8)
        pltpu.sync_copy(idx2.at[pl.ds(pl.multiple_of(base // _CH, 8), _NCHK)],
                        idx_v)

        for t, d in enumerate((d0, d1, d2, d3)):
            pltpu.sync_copy(zeros_hbm.at[pl.ds(rowo, _ROWS)],
                            acc.at[pl.ds(rowo, _ROWS)])
            plsc.subcore_barrier()

            def body(cc, carry, d=d):
                a = pl.multiple_of(base + cc * _CH, _CH)
                pltpu.sync_copy(d.at[pl.ds(a, _CH)], data_v)
                pltpu.sync_copy(data_v, acc.at[idx_v.at[cc]], add=True)
                return carry

            lax.fori_loop(0, _NCHK, body, 0)
            plsc.subcore_barrier()
            pltpu.sync_copy(acc.at[pl.ds(rowo, _ROWS)],
                            out_hbm.at[t, cid, pl.ds(rowo, _ROWS)])
            plsc.subcore_barrier()

    return k


def _sc_segsum4(d0, d1, d2, d3, dst, count):
    del count
    idx2 = dst.reshape(_E // _CH, _CH)
    zeros = jnp.zeros((_NP, _F), jnp.float32)
    out = _sc_segsum4_fn()(d0, d1, d2, d3, idx2, zeros)
    r = out[:, 0] + out[:, 1]
    return r[0, :_N], r[1, :_N], r[2, :_N], r[3, :_N]


def _silu(x):
    return x * jax.nn.sigmoid(x)


def _bessel(d):
    freqs = jnp.arange(1, _R + 1, dtype=jnp.float32) * math.pi / _CUT
    ax = d[:, None] * freqs[None, :]
    norm = jnp.where(d == 0, 1.0, d)
    return jnp.sin(ax) / norm[:, None]


def _cutoff_fn(d):
    return 0.5 * (jnp.cos(d * math.pi / _CUT) + 1.0) * (d < _CUT).astype(jnp.float32)


def kernel(z, pos, params):
    dst, src, valid, count = _radius_graph(pos)

    s = params["embedding"][z]
    v = jnp.zeros((3, _N, _F), dtype=jnp.float32)
    rij = pos[dst] - pos[src]
    d = jnp.linalg.norm(rij, axis=1)
    rbf = _bessel(d)
    cut = _cutoff_fn(d)
    rn = rij / jnp.maximum(d[:, None], 1e-12)

    for L in range(_L):
        mp = params["msg"][L]
        up = params["upd"][L]

        phi_n = _silu(s @ mp["lin1"]["W"] + mp["lin1"]["b"])
        phi_n = phi_n @ mp["lin2"]["W"] + mp["lin2"]["b"]
        W = (rbf @ mp["lin_rbf"]["W"] + mp["lin_rbf"]["b"]) * cut[:, None]
        pw = phi_n[src] * W
        pw = jnp.where(valid[:, None], pw, 0.0)
        left = pw[:, :_F]
        dsm = pw[:, _F:2 * _F]
        right = pw[:, 2 * _F:]
        dvm = [v[c][src] * left + right * rn[:, c:c + 1] for c in range(3)]
        ds, dv0, dv1, dv2 = _sc_segsum4(dsm, dvm[0], dvm[1], dvm[2], dst, count)
        s = ds + s
        v = v + jnp.stack([dv0, dv1, dv2], axis=0)

        U_v = jnp.einsum("cnf,fg->cng", v, up["denseU"]["W"])
        V_v = jnp.einsum("cnf,fg->cng", v, up["denseV"]["W"])
        dot = jnp.sum(U_v * V_v, axis=0)
        V_norm = jnp.sqrt(jnp.sum(V_v * V_v, axis=0) + 1e-12)
        a = jnp.concatenate([s, V_norm], axis=-1)
        a = _silu(a @ up["lin_up"]["W"] + up["lin_up"]["b"])
        a = a @ up["lin2"]["W"] + up["lin2"]["b"]
        a_vv = a[:, :_F]
        a_sv = a[:, _F:2 * _F]
        a_ss = a[:, 2 * _F:]
        s = s + a_ss + a_sv * dot
        v = v + U_v * a_vv[None, :, :]

    W = params["lin"]["W"]
    b = params["lin"]["b"]
    s = _silu(s @ W + b)
    s = s @ W + b
    return s
